# baseline (speedup 1.0000x reference)
import functools

import jax
import jax.numpy as jnp
from jax import lax
from jax.experimental import pallas as pl
from jax.experimental.pallas import tpu as pltpu
from jax.experimental.pallas import tpu_sc as plsc

_BB = 4


def _add_kernel(x_ref, t_ref, o_ref):
    o_ref[...] = x_ref[...] + t_ref[...][None]


def _tc_kernel(x, table):
    batch, num_patches, dim = x.shape
    grid = (batch // _BB,)
    return pl.pallas_call(
        _add_kernel,
        grid=grid,
        in_specs=[
            pl.BlockSpec((_BB, num_patches, dim), lambda b: (b, 0, 0)),
            pl.BlockSpec((num_patches, dim), lambda b: (0, 0)),
        ],
        out_specs=pl.BlockSpec((_BB, num_patches, dim), lambda b: (b, 0, 0)),
        out_shape=jax.ShapeDtypeStruct(x.shape, x.dtype),
        compiler_params=pltpu.CompilerParams(
            dimension_semantics=("parallel",),
        ),
    )(x, table)


_NC, _NS, _LANES = 2, 16, 16
_NW = _NC * _NS


def _sc_call(x2d, table):
    n_rows, dim = x2d.shape
    n_patches = table.shape[0]
    pw = n_patches // _NW
    n_batch = n_rows // n_patches
    mesh = plsc.VectorSubcoreMesh(core_axis_name="c", subcore_axis_name="s")

    @functools.partial(
        pl.kernel,
        out_type=jax.ShapeDtypeStruct((n_rows, dim), jnp.float32),
        mesh=mesh,
        scratch_types=[
            pltpu.VMEM((pw, dim), jnp.float32),
            pltpu.VMEM((pw, dim), jnp.float32),
        ],
    )
    def k(x_hbm, t_hbm, out_hbm, tv, xv):
        wid = lax.axis_index("s") * _NC + lax.axis_index("c")
        tbase = wid * pw
        pltpu.sync_copy(t_hbm.at[pl.ds(tbase, pw)], tv)

        @pl.loop(0, n_batch)
        def _body(b):
            off = b * n_patches + tbase
            pltpu.sync_copy(x_hbm.at[pl.ds(off, pw)], xv)

            @pl.loop(0, pw)
            def _row(i):
                for j in range(dim // _LANES):
                    sl = pl.ds(j * _LANES, _LANES)
                    xv[i, sl] = xv[i, sl] + tv[i, sl]

            pltpu.sync_copy(xv, out_hbm.at[pl.ds(off, pw)])

    return k(x2d, table)


def kernel(x, table):
    batch, num_patches, dim = x.shape
    out = _sc_call(x.reshape(batch * num_patches, dim), table)
    return out.reshape(batch, num_patches, dim)


# speedup vs baseline: 2.5219x; 2.5219x over previous
import jax
import jax.numpy as jnp
from jax.experimental import pallas as pl
from jax.experimental.pallas import tpu as pltpu

_BB = 4


def _add_kernel(x_ref, t_ref, o_ref):
    o_ref[...] = x_ref[...] + t_ref[...][None]


def kernel(x, table):
    batch, num_patches, dim = x.shape
    grid = (batch // _BB,)
    return pl.pallas_call(
        _add_kernel,
        grid=grid,
        in_specs=[
            pl.BlockSpec((_BB, num_patches, dim), lambda b: (b, 0, 0)),
            pl.BlockSpec((num_patches, dim), lambda b: (0, 0)),
        ],
        out_specs=pl.BlockSpec((_BB, num_patches, dim), lambda b: (b, 0, 0)),
        out_shape=jax.ShapeDtypeStruct(x.shape, x.dtype),
        compiler_params=pltpu.CompilerParams(
            dimension_semantics=("parallel",),
        ),
    )(x, table)


# speedup vs baseline: 2.5345x; 1.0050x over previous
import jax
import jax.numpy as jnp
from jax.experimental import pallas as pl
from jax.experimental.pallas import tpu as pltpu

_BB = 5


def _add_kernel(x_ref, t_ref, o_ref):
    o_ref[...] = x_ref[...] + t_ref[...][None]


def kernel(x, table):
    batch, num_patches, dim = x.shape
    grid = (pl.cdiv(batch, _BB),)
    return pl.pallas_call(
        _add_kernel,
        grid=grid,
        in_specs=[
            pl.BlockSpec((_BB, num_patches, dim), lambda b: (b, 0, 0)),
            pl.BlockSpec((num_patches, dim), lambda b: (0, 0)),
        ],
        out_specs=pl.BlockSpec((_BB, num_patches, dim), lambda b: (b, 0, 0)),
        out_shape=jax.ShapeDtypeStruct(x.shape, x.dtype),
        compiler_params=pltpu.CompilerParams(
            dimension_semantics=("parallel",),
            vmem_limit_bytes=120 * 1024 * 1024,
        ),
    )(x, table)
